# baseline (speedup 1.0000x reference)
import functools

import jax
import jax.numpy as jnp
from jax import lax
from jax.experimental import pallas as pl
from jax.experimental.pallas import tpu as pltpu
from jax.experimental.pallas import tpu_sc as plsc

VOCAB = 100000
DIM = 128
BATCH = 1024


def _gather_sc(table, idx):
    info = plsc.get_sparse_core_info()
    nw = info.num_cores * info.num_subcores
    bpw = BATCH // nw
    mesh = plsc.VectorSubcoreMesh(core_axis_name="c", subcore_axis_name="s")

    @functools.partial(
        pl.kernel,
        mesh=mesh,
        out_type=jax.ShapeDtypeStruct((BATCH, DIM), jnp.float32),
        scratch_types=[
            pltpu.VMEM((bpw,), jnp.int32),
            pltpu.VMEM((bpw, DIM), jnp.float32),
            pltpu.SemaphoreType.DMA,
        ],
    )
    def gather_kernel(table_hbm, idx_hbm, out_hbm, idx_v, rows_v, sem):
        wid = lax.axis_index("s") * info.num_cores + lax.axis_index("c")
        base = wid * bpw
        pltpu.sync_copy(idx_hbm.at[pl.ds(base, bpw)], idx_v)
        pltpu.async_copy(table_hbm.at[idx_v], rows_v, sem).wait()
        pltpu.sync_copy(rows_v, out_hbm.at[pl.ds(base, bpw)])

    return gather_kernel(table, idx)


_VT = 2048
_NT = (VOCAB + _VT - 1) // _VT
_NFULL = VOCAB // _VT
_TAIL = VOCAB - _NFULL * _VT
_NB = 5


def _proj_kernel(wt_ref, et_ref, b_ref, o_hbm, *scratch):
    bufs = scratch[:_NB]
    sems = scratch[_NB : 2 * _NB]
    buf_t, sem_t = scratch[2 * _NB :]
    j = pl.program_id(0)
    acc = jnp.dot(wt_ref[...], et_ref[...], preferred_element_type=jnp.float32)
    acc = acc + b_ref[...][0, :, None]

    @pl.when(j < _NFULL)
    def _():
        for s in range(_NB):

            @pl.when(lax.rem(j, _NB) == s)
            def _(s=s):
                @pl.when(j >= _NB)
                def _():
                    pltpu.make_async_copy(
                        bufs[s], o_hbm.at[pl.ds((j - _NB) * _VT, _VT), :], sems[s]
                    ).wait()

                bufs[s][...] = acc
                pltpu.make_async_copy(
                    bufs[s], o_hbm.at[pl.ds(j * _VT, _VT), :], sems[s]
                ).start()

    @pl.when(j == _NT - 1)
    def _():
        buf_t[...] = acc
        pltpu.make_async_copy(
            buf_t.at[pl.ds(0, _TAIL), :],
            o_hbm.at[pl.ds(_NFULL * _VT, _TAIL), :],
            sem_t,
        ).start()
        for s in range(_NB):
            jl = _NFULL - 1 - ((_NFULL - 1 - s) % _NB)
            pltpu.make_async_copy(
                bufs[s], o_hbm.at[pl.ds(jl * _VT, _VT), :], sems[s]
            ).wait()
        pltpu.make_async_copy(
            buf_t.at[pl.ds(0, _TAIL), :],
            o_hbm.at[pl.ds(_NFULL * _VT, _TAIL), :],
            sem_t,
        ).wait()


def _project_t(Wt, eT, b):
    b2 = b.reshape(1, VOCAB)
    return pl.pallas_call(
        _proj_kernel,
        grid=(_NT,),
        in_specs=[
            pl.BlockSpec((_VT, DIM), lambda j: (j, 0)),
            pl.BlockSpec((DIM, BATCH), lambda j: (0, 0)),
            pl.BlockSpec((1, _VT), lambda j: (0, j)),
        ],
        out_specs=pl.BlockSpec(memory_space=pl.ANY),
        out_shape=jax.ShapeDtypeStruct((VOCAB, BATCH), jnp.float32),
        scratch_shapes=(
            [pltpu.VMEM((_VT, BATCH), jnp.float32) for _ in range(_NB)]
            + [pltpu.SemaphoreType.DMA for _ in range(_NB)]
            + [pltpu.VMEM((_VT, BATCH), jnp.float32), pltpu.SemaphoreType.DMA]
        ),
    )(Wt, eT, b2)


def kernel(x, table, W, b):
    idx = x.astype(jnp.int32)
    e = _gather_sc(table, idx)
    out_t = _project_t(W.T, e.T, b)
    return out_t.T


# speedup vs baseline: 1.0132x; 1.0132x over previous
import functools

import jax
import jax.numpy as jnp
from jax import lax
from jax.experimental import pallas as pl
from jax.experimental.pallas import tpu as pltpu
from jax.experimental.pallas import tpu_sc as plsc

VOCAB = 100000
DIM = 128
BATCH = 1024


def _gather_sc(table, idx):
    info = plsc.get_sparse_core_info()
    nw = info.num_cores * info.num_subcores
    bpw = BATCH // nw
    mesh = plsc.VectorSubcoreMesh(core_axis_name="c", subcore_axis_name="s")

    @functools.partial(
        pl.kernel,
        mesh=mesh,
        out_type=jax.ShapeDtypeStruct((BATCH, DIM), jnp.float32),
        scratch_types=[
            pltpu.VMEM((bpw,), jnp.int32),
            pltpu.VMEM((bpw, DIM), jnp.float32),
            pltpu.SemaphoreType.DMA,
        ],
    )
    def gather_kernel(table_hbm, idx_hbm, out_hbm, idx_v, rows_v, sem):
        wid = lax.axis_index("s") * info.num_cores + lax.axis_index("c")
        base = wid * bpw
        pltpu.sync_copy(idx_hbm.at[pl.ds(base, bpw)], idx_v)
        pltpu.async_copy(table_hbm.at[idx_v], rows_v, sem).wait()
        pltpu.sync_copy(rows_v, out_hbm.at[pl.ds(base, bpw)])

    return gather_kernel(table, idx)


_VT = 2048
_NT = (VOCAB + _VT - 1) // _VT
_NFULL = VOCAB // _VT
_TAIL = VOCAB - _NFULL * _VT
_NB = 4


def _proj_kernel(wt_ref, e_ref, b_ref, o_hbm, *scratch):
    bufs = scratch[:_NB]
    sems = scratch[_NB : 2 * _NB]
    buf_t, sem_t, et_buf = scratch[2 * _NB :]
    j = pl.program_id(0)

    @pl.when(j == 0)
    def _():
        et_buf[...] = jnp.swapaxes(e_ref[...], 0, 1)

    acc = jnp.dot(wt_ref[...], et_buf[...], preferred_element_type=jnp.float32)
    acc = acc + b_ref[...][0, :, None]

    @pl.when(j < _NFULL)
    def _():
        for s in range(_NB):

            @pl.when(lax.rem(j, _NB) == s)
            def _(s=s):
                @pl.when(j >= _NB)
                def _():
                    pltpu.make_async_copy(
                        bufs[s], o_hbm.at[pl.ds((j - _NB) * _VT, _VT), :], sems[s]
                    ).wait()

                bufs[s][...] = acc
                pltpu.make_async_copy(
                    bufs[s], o_hbm.at[pl.ds(j * _VT, _VT), :], sems[s]
                ).start()

    @pl.when(j == _NT - 1)
    def _():
        buf_t[...] = acc
        pltpu.make_async_copy(
            buf_t.at[pl.ds(0, _TAIL), :],
            o_hbm.at[pl.ds(_NFULL * _VT, _TAIL), :],
            sem_t,
        ).start()
        for s in range(_NB):
            jl = _NFULL - 1 - ((_NFULL - 1 - s) % _NB)
            pltpu.make_async_copy(
                bufs[s], o_hbm.at[pl.ds(jl * _VT, _VT), :], sems[s]
            ).wait()
        pltpu.make_async_copy(
            buf_t.at[pl.ds(0, _TAIL), :],
            o_hbm.at[pl.ds(_NFULL * _VT, _TAIL), :],
            sem_t,
        ).wait()


def _project_t(Wt, e, b):
    b2 = b.reshape(1, VOCAB)
    return pl.pallas_call(
        _proj_kernel,
        grid=(_NT,),
        in_specs=[
            pl.BlockSpec((_VT, DIM), lambda j: (j, 0)),
            pl.BlockSpec((BATCH, DIM), lambda j: (0, 0)),
            pl.BlockSpec((1, _VT), lambda j: (0, j)),
        ],
        out_specs=pl.BlockSpec(memory_space=pl.ANY),
        out_shape=jax.ShapeDtypeStruct((VOCAB, BATCH), jnp.float32),
        scratch_shapes=(
            [pltpu.VMEM((_VT, BATCH), jnp.float32) for _ in range(_NB)]
            + [pltpu.SemaphoreType.DMA for _ in range(_NB)]
            + [
                pltpu.VMEM((_VT, BATCH), jnp.float32),
                pltpu.SemaphoreType.DMA,
                pltpu.VMEM((DIM, BATCH), jnp.float32),
            ]
        ),
    )(Wt, e, b2)


def kernel(x, table, W, b):
    idx = x.astype(jnp.int32)
    e = _gather_sc(table, idx)
    out_t = _project_t(W.T, e, b)
    return out_t.T


# speedup vs baseline: 1.0238x; 1.0105x over previous
import functools

import jax
import jax.numpy as jnp
from jax import lax
from jax.experimental import pallas as pl
from jax.experimental.pallas import tpu as pltpu
from jax.experimental.pallas import tpu_sc as plsc

VOCAB = 100000
DIM = 128
BATCH = 1024


def _gather_sc(table, idx):
    info = plsc.get_sparse_core_info()
    nw = info.num_cores * info.num_subcores
    bpw = BATCH // nw
    mesh = plsc.VectorSubcoreMesh(core_axis_name="c", subcore_axis_name="s")

    @functools.partial(
        pl.kernel,
        mesh=mesh,
        out_type=jax.ShapeDtypeStruct((BATCH, DIM), jnp.float32),
        scratch_types=[
            pltpu.VMEM((bpw,), jnp.int32),
            pltpu.VMEM((bpw, DIM), jnp.float32),
            pltpu.SemaphoreType.DMA,
        ],
    )
    def gather_kernel(table_hbm, idx_hbm, out_hbm, idx_v, rows_v, sem):
        wid = lax.axis_index("s") * info.num_cores + lax.axis_index("c")
        base = wid * bpw
        pltpu.sync_copy(idx_hbm.at[pl.ds(base, bpw)], idx_v)
        pltpu.async_copy(table_hbm.at[idx_v], rows_v, sem).wait()
        pltpu.sync_copy(rows_v, out_hbm.at[pl.ds(base, bpw)])

    return gather_kernel(table, idx)


_VT = 2048
_NT = (VOCAB + _VT - 1) // _VT
_NFULL = VOCAB // _VT
_TAIL = VOCAB - _NFULL * _VT
_NB = 4


def _proj_kernel(wt_ref, e_ref, b_ref, o_hbm, *scratch):
    bufs = scratch[:_NB]
    sems = scratch[_NB : 2 * _NB]
    buf_t, sem_t, et_buf = scratch[2 * _NB :]
    j = pl.program_id(0)

    @pl.when(j == 0)
    def _():
        et_buf[...] = jnp.swapaxes(e_ref[...], 0, 1)

    acc = jnp.dot(wt_ref[...], et_buf[...], preferred_element_type=jnp.float32)
    acc = acc + b_ref[...][0, :, None]

    @pl.when(j == 0)
    def _():
        buf_t[...] = acc
        pltpu.make_async_copy(
            buf_t.at[pl.ds(0, _TAIL), :],
            o_hbm.at[pl.ds(_NFULL * _VT, _TAIL), :],
            sem_t,
        ).start()

    @pl.when(j > 0)
    def _():
        jf = j - 1
        for s in range(_NB):

            @pl.when(lax.rem(jf, _NB) == s)
            def _(s=s):
                @pl.when(jf >= _NB)
                def _():
                    pltpu.make_async_copy(
                        bufs[s], o_hbm.at[pl.ds((jf - _NB) * _VT, _VT), :], sems[s]
                    ).wait()

                bufs[s][...] = acc
                pltpu.make_async_copy(
                    bufs[s], o_hbm.at[pl.ds(jf * _VT, _VT), :], sems[s]
                ).start()

    @pl.when(j == _NT - 1)
    def _():
        for s in range(_NB):
            jl = _NFULL - 1 - ((_NFULL - 1 - s) % _NB)
            pltpu.make_async_copy(
                bufs[s], o_hbm.at[pl.ds(jl * _VT, _VT), :], sems[s]
            ).wait()
        pltpu.make_async_copy(
            buf_t.at[pl.ds(0, _TAIL), :],
            o_hbm.at[pl.ds(_NFULL * _VT, _TAIL), :],
            sem_t,
        ).wait()


def _project_t(Wt, e, b):
    b2 = b.reshape(1, VOCAB)
    return pl.pallas_call(
        _proj_kernel,
        grid=(_NT,),
        in_specs=[
            pl.BlockSpec((_VT, DIM), lambda j: ((j + _NT - 1) % _NT, 0)),
            pl.BlockSpec((BATCH, DIM), lambda j: (0, 0)),
            pl.BlockSpec((1, _VT), lambda j: (0, (j + _NT - 1) % _NT)),
        ],
        out_specs=pl.BlockSpec(memory_space=pl.ANY),
        out_shape=jax.ShapeDtypeStruct((VOCAB, BATCH), jnp.float32),
        scratch_shapes=(
            [pltpu.VMEM((_VT, BATCH), jnp.float32) for _ in range(_NB)]
            + [pltpu.SemaphoreType.DMA for _ in range(_NB)]
            + [
                pltpu.VMEM((_VT, BATCH), jnp.float32),
                pltpu.SemaphoreType.DMA,
                pltpu.VMEM((DIM, BATCH), jnp.float32),
            ]
        ),
    )(Wt, e, b2)


def kernel(x, table, W, b):
    idx = x.astype(jnp.int32)
    e = _gather_sc(table, idx)
    out_t = _project_t(W.T, e, b)
    return out_t.T
